# baseline (speedup 1.0000x reference)
import functools

import jax
import jax.numpy as jnp
from jax import lax
from jax.experimental import pallas as pl
from jax.experimental.pallas import tpu as pltpu
from jax.experimental.pallas import tpu_sc as plsc

NUM_FIELDS = 26
FIELD_SIZE = 100000
EMBED_DIM = 16
LANES = 16

NC, NS = 2, 16
NW = NC * NS


def _embed_kernel(b_per_w, n_chunks, chunk, x_hbm, table_hbm, out_hbm,
                  idx_v, rows_v, sem):
    wid = lax.axis_index("s") * NC + lax.axis_index("c")

    pltpu.sync_copy(x_hbm.at[wid], idx_v)

    lane = lax.iota(jnp.int32, LANES)

    def add_offsets(j, c):
        pos = j * LANES + lane
        off = (pos % NUM_FIELDS) * FIELD_SIZE
        idx_v[c, pl.ds(j * LANES, LANES)] = idx_v[c, pl.ds(j * LANES, LANES)] + off
        return c

    for c in range(n_chunks):
        lax.fori_loop(0, chunk // LANES, add_offsets, c)

    out_base = wid * b_per_w
    for c in range(n_chunks):
        pltpu.async_copy(table_hbm.at[idx_v.at[c]], rows_v, sem).wait()
        pltpu.sync_copy(rows_v, out_hbm.at[pl.ds(out_base + c * chunk, chunk)])


def kernel(x, table):
    batch, num_fields = x.shape
    b_flat = batch * num_fields
    assert b_flat % NW == 0
    b_per_w = b_flat // NW
    n_chunks = 4
    chunk = b_per_w // n_chunks
    assert chunk % LANES == 0 and b_per_w % NUM_FIELDS == 0
    assert chunk % NUM_FIELDS == 0

    x_blocked = x.reshape(NW, n_chunks, chunk).astype(jnp.int32)

    mesh = plsc.VectorSubcoreMesh(core_axis_name="c", subcore_axis_name="s")
    k = functools.partial(
        pl.kernel,
        mesh=mesh,
        out_type=jax.ShapeDtypeStruct((b_flat, EMBED_DIM), jnp.float32),
        scratch_types=[
            pltpu.VMEM((n_chunks, chunk), jnp.int32),
            pltpu.VMEM((chunk, EMBED_DIM), jnp.float32),
            pltpu.SemaphoreType.DMA,
        ],
        compiler_params=pltpu.CompilerParams(use_tc_tiling_on_sc=False),
    )(functools.partial(_embed_kernel, b_per_w, n_chunks, chunk))

    out_flat = k(x_blocked, table)
    return out_flat.reshape(batch, num_fields, EMBED_DIM)


# speedup vs baseline: 1.0000x; 1.0000x over previous
import functools

import jax
import jax.numpy as jnp
from jax import lax
from jax.experimental import pallas as pl
from jax.experimental.pallas import tpu as pltpu
from jax.experimental.pallas import tpu_sc as plsc

NUM_FIELDS = 26
FIELD_SIZE = 100000
EMBED_DIM = 16
LANES = 16

NC, NS = 2, 16
NW = NC * NS


def _embed_kernel(b_per_w, n_chunks, chunk, x_hbm, table_hbm, out_hbm,
                  idx_v, rows_v, sem):
    wid = lax.axis_index("s") * NC + lax.axis_index("c")
    base = wid * b_per_w

    pltpu.sync_copy(x_hbm.at[pl.ds(base, b_per_w)], idx_v)

    lane = lax.iota(jnp.int32, LANES)

    def add_offsets(j, carry):
        pos = j * LANES + lane
        off = (pos % NUM_FIELDS) * FIELD_SIZE
        idx_v[pl.ds(j * LANES, LANES)] = idx_v[pl.ds(j * LANES, LANES)] + off
        return carry

    lax.fori_loop(0, b_per_w // LANES, add_offsets, 0)

    for c in range(n_chunks):
        pltpu.async_copy(
            table_hbm.at[idx_v.at[pl.ds(c * chunk, chunk)]], rows_v, sem
        ).wait()
        pltpu.sync_copy(rows_v, out_hbm.at[pl.ds(base + c * chunk, chunk)])


def kernel(x, table):
    batch, num_fields = x.shape
    b_flat = batch * num_fields
    assert b_flat % NW == 0
    b_per_w = b_flat // NW
    n_chunks = 4
    chunk = b_per_w // n_chunks
    assert chunk % LANES == 0 and b_per_w % NUM_FIELDS == 0

    x_flat = x.reshape(b_flat).astype(jnp.int32)

    mesh = plsc.VectorSubcoreMesh(core_axis_name="c", subcore_axis_name="s")
    k = functools.partial(
        pl.kernel,
        mesh=mesh,
        out_type=jax.ShapeDtypeStruct((b_flat, EMBED_DIM), jnp.float32),
        scratch_types=[
            pltpu.VMEM((b_per_w,), jnp.int32),
            pltpu.VMEM((chunk, EMBED_DIM), jnp.float32),
            pltpu.SemaphoreType.DMA,
        ],
        compiler_params=pltpu.CompilerParams(use_tc_tiling_on_sc=False),
    )(functools.partial(_embed_kernel, b_per_w, n_chunks, chunk))

    out_flat = k(x_flat, table)
    return out_flat.reshape(batch, num_fields, EMBED_DIM)


# speedup vs baseline: 1.2545x; 1.2545x over previous
import functools

import jax
import jax.numpy as jnp
from jax import lax
from jax.experimental import pallas as pl
from jax.experimental.pallas import tpu as pltpu
from jax.experimental.pallas import tpu_sc as plsc

NUM_FIELDS = 26
FIELD_SIZE = 100000
EMBED_DIM = 16
LANES = 16

NC, NS = 2, 16
NW = NC * NS

V = NUM_FIELDS * FIELD_SIZE
LT = (V + 127) // 128
FULL_LT = V // 128
TAIL = V - FULL_LT * 128
BLK_T = 8
N_BLK = FULL_LT // BLK_T
BLK_W = BLK_T * 128
V_PAD = LT * 128


def _detile_kernel(t_hbm, out_hbm, in_v, out_v, tail_in, tail_out):
    wid = lax.axis_index("s") * NC + lax.axis_index("c")
    lane = lax.iota(jnp.int32, LANES)

    def do_block(i, carry):
        blk = wid + i * NW

        @pl.when(blk < N_BLK)
        def _():
            pltpu.sync_copy(t_hbm.at[:, pl.ds(blk * BLK_W, BLK_W)], in_v)

            def do_tile(t, c2):
                col0 = t * 128
                for p in range(128):
                    col = jnp.full((LANES,), 0, jnp.int32) + (col0 + p)
                    vals = plsc.load_gather(in_v, [lane, col])
                    out_v[16 * t + (p // 8), pl.ds((p % 8) * 16, LANES)] = vals
                return c2

            lax.fori_loop(0, BLK_T, do_tile, 0)
            pltpu.sync_copy(out_v, out_hbm.at[pl.ds(blk * 128, 128)])

        return carry

    lax.fori_loop(0, (N_BLK + NW - 1) // NW, do_block, 0)

    @pl.when(wid == 0)
    def _():
        pltpu.sync_copy(t_hbm.at[:, pl.ds(FULL_LT * 128, TAIL)], tail_in)
        for p in range(TAIL):
            col = jnp.full((LANES,), p, jnp.int32)
            vals = plsc.load_gather(tail_in, [lane, col])
            tail_out[p // 8, pl.ds((p % 8) * 16, LANES)] = vals
        pltpu.sync_copy(tail_out, out_hbm.at[pl.ds(FULL_LT * 16, TAIL * 16 // 128)])


def _gather_kernel(b_per_w, n_chunks, chunk, x_hbm, t_hbm, out_hbm,
                   idx_v, rows_v, st_v, sem, osem):
    wid = lax.axis_index("s") * NC + lax.axis_index("c")
    base = wid * b_per_w
    lane = lax.iota(jnp.int32, LANES)

    pltpu.sync_copy(x_hbm.at[pl.ds(base, b_per_w)], idx_v)

    def add_offsets(j, carry):
        pos = j * LANES + lane
        off = (pos % NUM_FIELDS) * FIELD_SIZE
        idx_v[pl.ds(j * LANES, LANES)] = idx_v[pl.ds(j * LANES, LANES)] + off
        return carry

    lax.fori_loop(0, b_per_w // LANES, add_offsets, 0)

    lane26 = lane * NUM_FIELDS
    bpc = chunk // NUM_FIELDS

    for c in range(n_chunks):
        pltpu.async_copy(
            t_hbm.at[idx_v.at[pl.ds(c * chunk, chunk)]], rows_v, sem
        ).wait()

        def retile(fm, carry):
            f = fm // (bpc // LANES)
            m = fm % (bpc // LANES)
            row = lane26 + (m * (LANES * NUM_FIELDS) + f)
            for st in range(2):
                for s in range(8):
                    e = jnp.full((LANES,), 8 * st + s, jnp.int32)
                    vals = plsc.load_gather(rows_v, [row, e])
                    st_v[f, st, s, pl.ds(m * LANES, LANES)] = vals
            return carry

        lax.fori_loop(0, NUM_FIELDS * (bpc // LANES), retile, 0)

        bt = wid * n_chunks + c
        copies = []
        for f in range(NUM_FIELDS):
            for st in range(2):
                copies.append(
                    pltpu.async_copy(st_v.at[f, st], out_hbm.at[f, st, bt], osem)
                )
        for cp in copies:
            cp.wait()


def kernel(x, table):
    batch, num_fields = x.shape
    b_flat = batch * num_fields
    b_per_w = b_flat // NW
    n_chunks = 4
    chunk = b_per_w // n_chunks

    mesh = plsc.VectorSubcoreMesh(core_axis_name="c", subcore_axis_name="s")

    t_nat = table.T
    detile = functools.partial(
        pl.kernel,
        mesh=mesh,
        out_type=jax.ShapeDtypeStruct((LT * 16, 128), jnp.float32),
        scratch_types=[
            pltpu.VMEM((LANES, BLK_W), jnp.float32),
            pltpu.VMEM((128, 128), jnp.float32),
            pltpu.VMEM((LANES, TAIL), jnp.float32),
            pltpu.VMEM((TAIL * 16 // 128, 128), jnp.float32),
        ],
        compiler_params=pltpu.CompilerParams(
            use_tc_tiling_on_sc=True, needs_layout_passes=False),
    )(_detile_kernel)
    t128 = detile(t_nat)
    t_lin = t128.reshape(V_PAD, EMBED_DIM)

    x_flat = x.reshape(b_flat).astype(jnp.int32)
    gather = functools.partial(
        pl.kernel,
        mesh=mesh,
        out_type=jax.ShapeDtypeStruct(
            (NUM_FIELDS, 2, batch // 128, 8, 128), jnp.float32),
        scratch_types=[
            pltpu.VMEM((b_per_w,), jnp.int32),
            pltpu.VMEM((chunk, EMBED_DIM), jnp.float32),
            pltpu.VMEM((NUM_FIELDS, 2, 8, 128), jnp.float32),
            pltpu.SemaphoreType.DMA,
            pltpu.SemaphoreType.DMA,
        ],
        compiler_params=pltpu.CompilerParams(
            use_tc_tiling_on_sc=False, needs_layout_passes=False),
    )(functools.partial(_gather_kernel, b_per_w, n_chunks, chunk))
    out5d = gather(x_flat, t_lin)

    return out5d.transpose(2, 4, 0, 1, 3).reshape(batch, num_fields, EMBED_DIM)


# speedup vs baseline: 1.3223x; 1.0540x over previous
import functools

import jax
import jax.numpy as jnp
from jax import lax
from jax.experimental import pallas as pl
from jax.experimental.pallas import tpu as pltpu
from jax.experimental.pallas import tpu_sc as plsc

NUM_FIELDS = 26
FIELD_SIZE = 100000
EMBED_DIM = 16
LANES = 16

NC, NS = 2, 16
NW = NC * NS

V = NUM_FIELDS * FIELD_SIZE
LT = (V + 127) // 128
FULL_LT = V // 128
TAIL = V - FULL_LT * 128
BLK_T = 8
N_BLK = FULL_LT // BLK_T
BLK_W = BLK_T * 128
V_PAD = LT * 128


def _detile_kernel(t_hbm, out_hbm, in_v, out_v, tail_in, tail_out,
                   sem_in, sem_out):
    wid = lax.axis_index("s") * NC + lax.axis_index("c")
    lane = lax.iota(jnp.int32, LANES)
    n_i = (N_BLK + NW - 1) // NW

    @pl.when(wid < N_BLK)
    def _():
        pltpu.async_copy(t_hbm.at[:, pl.ds(wid * BLK_W, BLK_W)],
                         in_v.at[0], sem_in)

    def do_block(i, carry):
        blk = wid + i * NW
        par = lax.rem(i, 2)

        @pl.when(blk < N_BLK)
        def _():
            pltpu.make_async_copy(
                t_hbm.at[:, pl.ds(0, BLK_W)], in_v.at[par], sem_in).wait()

            nblk = blk + NW

            @pl.when(nblk < N_BLK)
            def _():
                pltpu.async_copy(
                    t_hbm.at[:, pl.ds(nblk * BLK_W, BLK_W)],
                    in_v.at[1 - par], sem_in)

            @pl.when(i >= 2)
            def _():
                pltpu.make_async_copy(
                    out_v.at[par], out_hbm.at[pl.ds(0, 128)], sem_out).wait()

            def do_tile(t, c2):
                col0 = t * 128
                for p in range(128):
                    col = jnp.full((LANES,), col0 + p, jnp.int32)
                    vals = plsc.load_gather(in_v.at[par], [lane, col])
                    out_v[par, 16 * t + (p // 8),
                          pl.ds((p % 8) * LANES, LANES)] = vals
                return c2

            lax.fori_loop(0, BLK_T, do_tile, 0)
            pltpu.async_copy(out_v.at[par],
                             out_hbm.at[pl.ds(blk * 128, 128)], sem_out)

        return carry

    lax.fori_loop(0, n_i, do_block, 0)

    nb = lax.max((N_BLK - 1 - wid) // NW + 1, 0)
    for k in range(2):
        @pl.when(nb >= k + 1)
        def _():
            pltpu.make_async_copy(
                out_v.at[0], out_hbm.at[pl.ds(0, 128)], sem_out).wait()

    @pl.when(wid == 0)
    def _():
        pltpu.sync_copy(t_hbm.at[:, pl.ds(FULL_LT * 128, TAIL)], tail_in)
        for p in range(TAIL):
            col = jnp.full((LANES,), p, jnp.int32)
            vals = plsc.load_gather(tail_in, [lane, col])
            tail_out[p // 8, pl.ds((p % 8) * LANES, LANES)] = vals
        pltpu.sync_copy(tail_out,
                        out_hbm.at[pl.ds(FULL_LT * 16, TAIL * 16 // 128)])


def _gather_kernel(b_per_w, n_chunks, chunk, x_hbm, t_hbm, out_hbm,
                   idx_v, rows_v, st_v, sem, osem):
    wid = lax.axis_index("s") * NC + lax.axis_index("c")
    base = wid * b_per_w
    lane = lax.iota(jnp.int32, LANES)

    pltpu.sync_copy(x_hbm.at[pl.ds(base, b_per_w)], idx_v)

    def add_offsets(j, carry):
        pos = j * LANES + lane
        off = (pos % NUM_FIELDS) * FIELD_SIZE
        idx_v[pl.ds(j * LANES, LANES)] = idx_v[pl.ds(j * LANES, LANES)] + off
        return carry

    lax.fori_loop(0, b_per_w // LANES, add_offsets, 0)

    mapvec = (lane // 8) * 1024 + (lane % 8) * 128
    bpc = chunk // NUM_FIELDS

    for c in range(n_chunks):
        pltpu.async_copy(
            t_hbm.at[idx_v.at[pl.ds(c * chunk, chunk)]], rows_v, sem
        ).wait()

        def retile(b1, carry):
            r0 = b1 * NUM_FIELDS
            for f in range(NUM_FIELDS):
                vals = rows_v[r0 + f]
                plsc.store_scatter(st_v, [mapvec + (b1 + 2048 * f)], vals)
            return carry

        lax.fori_loop(0, bpc, retile, 0)

        bt = wid * n_chunks + c
        copies = []
        for f in range(NUM_FIELDS):
            for st in range(2):
                copies.append(pltpu.async_copy(
                    st_v.at[pl.ds(f * 2048 + st * 1024, 1024)],
                    out_hbm.at[f, st, bt], osem))
        for cp in copies:
            cp.wait()


def kernel(x, table):
    batch, num_fields = x.shape
    b_flat = batch * num_fields
    b_per_w = b_flat // NW
    n_chunks = 4
    chunk = b_per_w // n_chunks

    mesh = plsc.VectorSubcoreMesh(core_axis_name="c", subcore_axis_name="s")

    t_nat = table.T
    detile = functools.partial(
        pl.kernel,
        mesh=mesh,
        out_type=jax.ShapeDtypeStruct((LT * 16, 128), jnp.float32),
        scratch_types=[
            pltpu.VMEM((2, LANES, BLK_W), jnp.float32),
            pltpu.VMEM((2, 128, 128), jnp.float32),
            pltpu.VMEM((LANES, TAIL), jnp.float32),
            pltpu.VMEM((TAIL * 16 // 128, 128), jnp.float32),
            pltpu.SemaphoreType.DMA,
            pltpu.SemaphoreType.DMA,
        ],
        compiler_params=pltpu.CompilerParams(
            use_tc_tiling_on_sc=True, needs_layout_passes=False),
    )(_detile_kernel)
    t128 = detile(t_nat)
    t_lin = t128.reshape(V_PAD, EMBED_DIM)

    x_flat = x.reshape(b_flat).astype(jnp.int32)
    gather = functools.partial(
        pl.kernel,
        mesh=mesh,
        out_type=jax.ShapeDtypeStruct(
            (NUM_FIELDS, 2, batch // 128, 8 * 128), jnp.float32),
        scratch_types=[
            pltpu.VMEM((b_per_w,), jnp.int32),
            pltpu.VMEM((chunk, EMBED_DIM), jnp.float32),
            pltpu.VMEM((NUM_FIELDS * 2 * 1024,), jnp.float32),
            pltpu.SemaphoreType.DMA,
            pltpu.SemaphoreType.DMA,
        ],
        compiler_params=pltpu.CompilerParams(
            use_tc_tiling_on_sc=False, needs_layout_passes=False),
    )(functools.partial(_gather_kernel, b_per_w, n_chunks, chunk))
    out4d = gather(x_flat, t_lin)

    out5d = out4d.reshape(NUM_FIELDS, 2, batch // 128, 8, 128)
    return out5d.transpose(2, 4, 0, 1, 3).reshape(batch, num_fields, EMBED_DIM)


# speedup vs baseline: 2.6271x; 1.9868x over previous
import functools

import jax
import jax.numpy as jnp
from jax import lax
from jax.experimental import pallas as pl
from jax.experimental.pallas import tpu as pltpu
from jax.experimental.pallas import tpu_sc as plsc

NUM_FIELDS = 26
FIELD_SIZE = 100000
EMBED_DIM = 16
LANES = 16

NC, NS = 2, 16
NW = NC * NS

V = NUM_FIELDS * FIELD_SIZE
LT = (V + 127) // 128
FULL_LT = V // 128
TAIL = V - FULL_LT * 128
BLK_T = 8
N_BLK = FULL_LT // BLK_T
BLK_W = BLK_T * 128
V_PAD = LT * 128


BLK_WORDS = 128 * 128


def _detile_kernel(t_hbm, out_hbm, in_v0, in_v1, ov0, ov1, tail_in, tail_out,
                   sem_in, sem_out):
    wid = lax.axis_index("s") * NC + lax.axis_index("c")
    lane = lax.iota(jnp.int32, LANES)
    pvec = (lane // 8) * 128 + (lane % 8) * LANES
    n_i2 = (N_BLK + 2 * NW - 1) // (2 * NW)

    @pl.when(wid < N_BLK)
    def _():
        pltpu.async_copy(t_hbm.at[:, pl.ds(wid * BLK_W, BLK_W)], in_v0, sem_in)

    def process(iv, ov, blk, have_prev_out):
        @pl.when(blk < N_BLK)
        def _():
            pltpu.make_async_copy(
                t_hbm.at[:, pl.ds(0, BLK_W)], iv, sem_in).wait()

            nblk = blk + NW
            nxt = in_v1 if iv is in_v0 else in_v0

            @pl.when(nblk < N_BLK)
            def _():
                pltpu.async_copy(
                    t_hbm.at[:, pl.ds(nblk * BLK_W, BLK_W)], nxt, sem_in)

            @pl.when(have_prev_out)
            def _():
                pltpu.make_async_copy(
                    ov, out_hbm.at[pl.ds(0, BLK_WORDS)], sem_out).wait()

            def do_tile(t, c2):
                base = pvec + t * 2048
                for e in range(16):
                    for q in range(8):
                        vals = iv[e, pl.ds(t * 128 + q * LANES, LANES)]
                        plsc.store_scatter(ov, [base + (q * 256 + e)], vals)
                return c2

            lax.fori_loop(0, BLK_T, do_tile, 0)
            pltpu.async_copy(ov, out_hbm.at[pl.ds(blk * BLK_WORDS, BLK_WORDS)],
                             sem_out)

    def do_pair(i, carry):
        blk0 = wid + (2 * i) * NW
        process(in_v0, ov0, blk0, i >= 1)
        process(in_v1, ov1, blk0 + NW, i >= 1)
        return carry

    lax.fori_loop(0, n_i2, do_pair, 0)

    nb = lax.max((N_BLK - 1 - wid) // NW + 1, 0)
    for k in range(2):
        @pl.when(nb >= k + 1)
        def _():
            pltpu.make_async_copy(
                ov0, out_hbm.at[pl.ds(0, BLK_WORDS)], sem_out).wait()

    @pl.when(wid == 0)
    def _():
        pltpu.sync_copy(t_hbm.at[:, pl.ds(FULL_LT * 128, TAIL)], tail_in)
        for p in range(TAIL):
            col = jnp.full((LANES,), p, jnp.int32)
            vals = plsc.load_gather(tail_in, [lane, col])
            plsc.store_scatter(
                tail_out, [lane + (128 * (p // 8) + LANES * (p % 8))], vals)
        pltpu.sync_copy(tail_out,
                        out_hbm.at[pl.ds(N_BLK * BLK_WORDS, TAIL * 16)])


def _gather_kernel(b_per_w, n_chunks, chunk, x_hbm, t_hbm, out_hbm,
                   idx_v, rows_v, st_v, sem, osem):
    wid = lax.axis_index("s") * NC + lax.axis_index("c")
    base = wid * b_per_w
    lane = lax.iota(jnp.int32, LANES)

    pltpu.sync_copy(x_hbm.at[pl.ds(base, b_per_w)], idx_v)

    def add_offsets(j, carry):
        pos = j * LANES + lane
        off = (pos % NUM_FIELDS) * FIELD_SIZE
        idx_v[pl.ds(j * LANES, LANES)] = idx_v[pl.ds(j * LANES, LANES)] + off
        return carry

    lax.fori_loop(0, b_per_w // LANES, add_offsets, 0)

    mapvec = (lane // 8) * 1024 + (lane % 8) * 128
    bpc = chunk // NUM_FIELDS

    for c in range(n_chunks):
        pltpu.async_copy(
            t_hbm.at[idx_v.at[pl.ds(c * chunk, chunk)]], rows_v, sem
        ).wait()

        def retile(b1, carry):
            r0 = b1 * NUM_FIELDS
            for f in range(NUM_FIELDS):
                vals = rows_v[r0 + f]
                plsc.store_scatter(st_v, [mapvec + (b1 + 2048 * f)], vals)
            return carry

        lax.fori_loop(0, bpc, retile, 0)

        bt = wid * n_chunks + c
        copies = []
        for f in range(NUM_FIELDS):
            for st in range(2):
                copies.append(pltpu.async_copy(
                    st_v.at[pl.ds(f * 2048 + st * 1024, 1024)],
                    out_hbm.at[f, st, bt], osem))
        for cp in copies:
            cp.wait()


def kernel(x, table):
    batch, num_fields = x.shape
    b_flat = batch * num_fields
    b_per_w = b_flat // NW
    n_chunks = 4
    chunk = b_per_w // n_chunks

    mesh = plsc.VectorSubcoreMesh(core_axis_name="c", subcore_axis_name="s")

    t_nat = table.T
    detile = functools.partial(
        pl.kernel,
        mesh=mesh,
        out_type=jax.ShapeDtypeStruct((V_PAD * EMBED_DIM,), jnp.float32),
        scratch_types=[
            pltpu.VMEM((LANES, BLK_W), jnp.float32),
            pltpu.VMEM((LANES, BLK_W), jnp.float32),
            pltpu.VMEM((BLK_WORDS,), jnp.float32),
            pltpu.VMEM((BLK_WORDS,), jnp.float32),
            pltpu.VMEM((LANES, TAIL), jnp.float32),
            pltpu.VMEM((TAIL * 16,), jnp.float32),
            pltpu.SemaphoreType.DMA,
            pltpu.SemaphoreType.DMA,
        ],
        compiler_params=pltpu.CompilerParams(
            use_tc_tiling_on_sc=True, needs_layout_passes=False),
    )(_detile_kernel)
    t128 = detile(t_nat)
    t_lin = t128.reshape(V_PAD, EMBED_DIM)

    x_flat = x.reshape(b_flat).astype(jnp.int32)
    gather = functools.partial(
        pl.kernel,
        mesh=mesh,
        out_type=jax.ShapeDtypeStruct(
            (NUM_FIELDS, 2, batch // 128, 8 * 128), jnp.float32),
        scratch_types=[
            pltpu.VMEM((b_per_w,), jnp.int32),
            pltpu.VMEM((chunk, EMBED_DIM), jnp.float32),
            pltpu.VMEM((NUM_FIELDS * 2 * 1024,), jnp.float32),
            pltpu.SemaphoreType.DMA,
            pltpu.SemaphoreType.DMA,
        ],
        compiler_params=pltpu.CompilerParams(
            use_tc_tiling_on_sc=False, needs_layout_passes=False),
    )(functools.partial(_gather_kernel, b_per_w, n_chunks, chunk))
    out4d = gather(x_flat, t_lin)

    out5d = out4d.reshape(NUM_FIELDS, 2, batch // 128, 8, 128)
    return out5d.transpose(2, 4, 0, 1, 3).reshape(batch, num_fields, EMBED_DIM)


# speedup vs baseline: 2.6346x; 1.0029x over previous
import functools

import jax
import jax.numpy as jnp
from jax import lax
from jax.experimental import pallas as pl
from jax.experimental.pallas import tpu as pltpu
from jax.experimental.pallas import tpu_sc as plsc

NUM_FIELDS = 26
FIELD_SIZE = 100000
EMBED_DIM = 16
LANES = 16

NC, NS = 2, 16
NW = NC * NS

V = NUM_FIELDS * FIELD_SIZE
LT = (V + 127) // 128
FULL_LT = V // 128
TAIL = V - FULL_LT * 128
BLK_T = 8
N_BLK = FULL_LT // BLK_T
BLK_W = BLK_T * 128
V_PAD = LT * 128


BLK_WORDS = 128 * 128


def _detile_kernel(t_hbm, out_hbm, in_v0, in_v1, ov0, ov1, tail_in, tail_out,
                   sem_in, sem_out):
    wid = lax.axis_index("s") * NC + lax.axis_index("c")
    lane = lax.iota(jnp.int32, LANES)
    pvec = (lane // 8) * 128 + (lane % 8) * LANES
    n_i2 = (N_BLK + 2 * NW - 1) // (2 * NW)

    @pl.when(wid < N_BLK)
    def _():
        pltpu.async_copy(t_hbm.at[:, pl.ds(wid * BLK_W, BLK_W)], in_v0, sem_in)

    def process(iv, ov, blk, have_prev_out):
        @pl.when(blk < N_BLK)
        def _():
            pltpu.make_async_copy(
                t_hbm.at[:, pl.ds(0, BLK_W)], iv, sem_in).wait()

            nblk = blk + NW
            nxt = in_v1 if iv is in_v0 else in_v0

            @pl.when(nblk < N_BLK)
            def _():
                pltpu.async_copy(
                    t_hbm.at[:, pl.ds(nblk * BLK_W, BLK_W)], nxt, sem_in)

            @pl.when(have_prev_out)
            def _():
                pltpu.make_async_copy(
                    ov, out_hbm.at[pl.ds(0, BLK_WORDS)], sem_out).wait()

            for t in range(BLK_T):
                base = pvec + t * 2048
                for e in range(16):
                    for q in range(8):
                        vals = iv[e, pl.ds(t * 128 + q * LANES, LANES)]
                        plsc.store_scatter(ov, [base + (q * 256 + e)], vals)
            pltpu.async_copy(ov, out_hbm.at[pl.ds(blk * BLK_WORDS, BLK_WORDS)],
                             sem_out)

    def do_pair(i, carry):
        blk0 = wid + (2 * i) * NW
        process(in_v0, ov0, blk0, i >= 1)
        process(in_v1, ov1, blk0 + NW, i >= 1)
        return carry

    lax.fori_loop(0, n_i2, do_pair, 0)

    nb = lax.max((N_BLK - 1 - wid) // NW + 1, 0)
    for k in range(2):
        @pl.when(nb >= k + 1)
        def _():
            pltpu.make_async_copy(
                ov0, out_hbm.at[pl.ds(0, BLK_WORDS)], sem_out).wait()

    @pl.when(wid == 0)
    def _():
        pltpu.sync_copy(t_hbm.at[:, pl.ds(FULL_LT * 128, TAIL)], tail_in)
        for p in range(TAIL):
            col = jnp.full((LANES,), p, jnp.int32)
            vals = plsc.load_gather(tail_in, [lane, col])
            plsc.store_scatter(
                tail_out, [lane + (128 * (p // 8) + LANES * (p % 8))], vals)
        pltpu.sync_copy(tail_out,
                        out_hbm.at[pl.ds(N_BLK * BLK_WORDS, TAIL * 16)])


def _gather_kernel(b_per_w, n_chunks, chunk, x_hbm, t_hbm, out_hbm,
                   idx_v, rows_v, st_v, sem, osem):
    wid = lax.axis_index("s") * NC + lax.axis_index("c")
    base = wid * b_per_w
    lane = lax.iota(jnp.int32, LANES)

    pltpu.sync_copy(x_hbm.at[pl.ds(base, b_per_w)], idx_v)

    def add_offsets(j, carry):
        pos = j * LANES + lane
        off = (pos % NUM_FIELDS) * FIELD_SIZE
        idx_v[pl.ds(j * LANES, LANES)] = idx_v[pl.ds(j * LANES, LANES)] + off
        return carry

    lax.fori_loop(0, b_per_w // LANES, add_offsets, 0)

    mapvec = (lane // 8) * 1024 + (lane % 8) * 128
    bpc = chunk // NUM_FIELDS

    pltpu.async_copy(t_hbm.at[idx_v.at[pl.ds(0, chunk)]], rows_v, sem)

    for c in range(n_chunks):
        pltpu.make_async_copy(
            t_hbm.at[idx_v.at[pl.ds(0, chunk)]], rows_v, sem).wait()

        def retile(b2, carry):
            for db in range(4):
                b1 = b2 * 4 + db
                r0 = b1 * NUM_FIELDS
                for f in range(NUM_FIELDS):
                    vals = rows_v[r0 + f]
                    plsc.store_scatter(st_v, [mapvec + (b1 + 2048 * f)], vals)
            return carry

        lax.fori_loop(0, bpc // 4, retile, 0)

        if c + 1 < n_chunks:
            pltpu.async_copy(
                t_hbm.at[idx_v.at[pl.ds((c + 1) * chunk, chunk)]], rows_v, sem)

        bt = wid * n_chunks + c
        copies = []
        for f in range(NUM_FIELDS):
            for st in range(2):
                copies.append(pltpu.async_copy(
                    st_v.at[pl.ds(f * 2048 + st * 1024, 1024)],
                    out_hbm.at[f, st, bt], osem))
        for cp in copies:
            cp.wait()


def kernel(x, table):
    batch, num_fields = x.shape
    b_flat = batch * num_fields
    b_per_w = b_flat // NW
    n_chunks = 4
    chunk = b_per_w // n_chunks

    mesh = plsc.VectorSubcoreMesh(core_axis_name="c", subcore_axis_name="s")

    t_nat = table.T
    detile = functools.partial(
        pl.kernel,
        mesh=mesh,
        out_type=jax.ShapeDtypeStruct((V_PAD * EMBED_DIM,), jnp.float32),
        scratch_types=[
            pltpu.VMEM((LANES, BLK_W), jnp.float32),
            pltpu.VMEM((LANES, BLK_W), jnp.float32),
            pltpu.VMEM((BLK_WORDS,), jnp.float32),
            pltpu.VMEM((BLK_WORDS,), jnp.float32),
            pltpu.VMEM((LANES, TAIL), jnp.float32),
            pltpu.VMEM((TAIL * 16,), jnp.float32),
            pltpu.SemaphoreType.DMA,
            pltpu.SemaphoreType.DMA,
        ],
        compiler_params=pltpu.CompilerParams(
            use_tc_tiling_on_sc=True, needs_layout_passes=False),
    )(_detile_kernel)
    t128 = detile(t_nat)
    t_lin = t128.reshape(V_PAD, EMBED_DIM)

    x_flat = x.reshape(b_flat).astype(jnp.int32)
    gather = functools.partial(
        pl.kernel,
        mesh=mesh,
        out_type=jax.ShapeDtypeStruct(
            (NUM_FIELDS, 2, batch // 128, 8 * 128), jnp.float32),
        scratch_types=[
            pltpu.VMEM((b_per_w,), jnp.int32),
            pltpu.VMEM((chunk, EMBED_DIM), jnp.float32),
            pltpu.VMEM((NUM_FIELDS * 2 * 1024,), jnp.float32),
            pltpu.SemaphoreType.DMA,
            pltpu.SemaphoreType.DMA,
        ],
        compiler_params=pltpu.CompilerParams(
            use_tc_tiling_on_sc=False, needs_layout_passes=False),
    )(functools.partial(_gather_kernel, b_per_w, n_chunks, chunk))
    out4d = gather(x_flat, t_lin)

    out5d = out4d.reshape(NUM_FIELDS, 2, batch // 128, 8, 128)
    return out5d.transpose(2, 4, 0, 1, 3).reshape(batch, num_fields, EMBED_DIM)
